# Initial kernel scaffold; baseline (speedup 1.0000x reference)
import jax
import jax.numpy as jnp
from jax.experimental import pallas as pl


def kernel(indices, weight):
    raise NotImplementedError("write your pallas kernel here")


# speedup vs baseline: 5.1808x; 5.1808x over previous
import functools

import jax
import jax.numpy as jnp
from jax import lax
from jax.experimental import pallas as pl
from jax.experimental.pallas import tpu as pltpu
from jax.experimental.pallas import tpu_sc as plsc

VOCAB = 100000
EMBED = 128
B_TOTAL = 4096 * 200
NC, NS = 2, 16
NW = NC * NS
B_PER_W = B_TOTAL // NW
CHUNK = 128
N_CHUNKS = B_PER_W // CHUNK

_mesh = plsc.VectorSubcoreMesh(core_axis_name="c", subcore_axis_name="s")


@functools.partial(
    pl.kernel,
    mesh=_mesh,
    out_type=jax.ShapeDtypeStruct((B_TOTAL, EMBED), jnp.float32),
    scratch_types=[
        pltpu.VMEM((CHUNK,), jnp.int32),
        pltpu.VMEM((CHUNK, EMBED), jnp.float32),
        pltpu.SemaphoreType.DMA,
    ],
)
def _embed_sc(idx_hbm, w_hbm, out_hbm, idx_v, rows_v, sem):
    wid = lax.axis_index("s") * NC + lax.axis_index("c")
    base = wid * B_PER_W

    def body(i, _):
        off = base + i * CHUNK
        pltpu.sync_copy(idx_hbm.at[pl.ds(off, CHUNK)], idx_v)
        pltpu.async_copy(w_hbm.at[idx_v], rows_v, sem).wait()
        pltpu.sync_copy(rows_v, out_hbm.at[pl.ds(off, CHUNK)])
        return 0

    lax.fori_loop(0, N_CHUNKS, body, 0)


def kernel(indices, weight):
    idx_flat = indices.reshape(-1).astype(jnp.int32)
    out = _embed_sc(idx_flat, weight)
    return out.reshape(indices.shape + (EMBED,))


# speedup vs baseline: 9.2391x; 1.7833x over previous
import functools

import jax
import jax.numpy as jnp
from jax import lax
from jax.experimental import pallas as pl
from jax.experimental.pallas import tpu as pltpu
from jax.experimental.pallas import tpu_sc as plsc

VOCAB = 100000
EMBED = 128
B_TOTAL = 4096 * 200
NC, NS = 2, 16
NW = NC * NS
B_PER_W = B_TOTAL // NW
CHUNK = 128
N_CHUNKS = B_PER_W // CHUNK
NBUF = 4
D = 2
NG = N_CHUNKS // NBUF

_mesh = plsc.VectorSubcoreMesh(core_axis_name="c", subcore_axis_name="s")


@functools.partial(
    pl.kernel,
    mesh=_mesh,
    out_type=jax.ShapeDtypeStruct((B_TOTAL, EMBED), jnp.float32),
    scratch_types=[
        pltpu.VMEM((N_CHUNKS, CHUNK), jnp.int32),
        pltpu.VMEM((NBUF, CHUNK, EMBED), jnp.float32),
        pltpu.SemaphoreType.DMA((NBUF,)),
        pltpu.SemaphoreType.DMA((NBUF,)),
    ],
)
def _embed_sc(idx_hbm, w_hbm, out_hbm, idx_v, rows_v, gsem, wsem):
    wid = lax.axis_index("s") * NC + lax.axis_index("c")
    base = wid * B_PER_W
    pltpu.sync_copy(idx_hbm.at[wid], idx_v)

    def fire_gather(g, b):
        pltpu.async_copy(w_hbm.at[idx_v.at[g]], rows_v.at[b], gsem.at[b])

    def wait_gather(g, b):
        pltpu.make_async_copy(w_hbm.at[idx_v.at[g]], rows_v.at[b],
                              gsem.at[b]).wait()

    def fire_wb(g, b):
        pltpu.async_copy(rows_v.at[b],
                         out_hbm.at[pl.ds(base + g * CHUNK, CHUNK)],
                         wsem.at[b])

    def wait_wb(g, b):
        pltpu.make_async_copy(rows_v.at[b],
                              out_hbm.at[pl.ds(base + g * CHUNK, CHUNK)],
                              wsem.at[b]).wait()

    def step(g, b, first, last):
        gg = g + D
        bb = (b + D) % NBUF
        if not last:
            if not first:
                wait_wb(gg - NBUF, bb)
            fire_gather(gg, bb)
        wait_gather(g, b)
        fire_wb(g, b)

    for b in range(D):
        fire_gather(b, b)
    for b in range(NBUF):
        step(b, b, first=(b + D < NBUF), last=False)

    def group(k, _):
        for b in range(NBUF):
            step(k * NBUF + b, b, first=False, last=False)
        return 0

    lax.fori_loop(1, NG - 1, group, 0)

    for b in range(NBUF):
        g = (NG - 1) * NBUF + b
        step(g, b, first=False, last=(g + D >= N_CHUNKS))
    for b in range(NBUF):
        wait_wb((NG - 1) * NBUF + b, b)


def kernel(indices, weight):
    idx = indices.reshape(NW, N_CHUNKS, CHUNK).astype(jnp.int32)
    out = _embed_sc(idx, weight)
    return out.reshape(indices.shape + (EMBED,))
